# Initial kernel scaffold; baseline (speedup 1.0000x reference)
import jax
import jax.numpy as jnp
from jax.experimental import pallas as pl


def kernel(state, embedding, state_min, state_max):
    raise NotImplementedError("write your pallas kernel here")


# speedup vs baseline: 3.8983x; 3.8983x over previous
import jax
import jax.numpy as jnp
from jax import lax
from jax.experimental import pallas as pl
from jax.experimental.pallas import tpu as pltpu
from jax.experimental.pallas import tpu_sc as plsc

_STATE_DIM = 64
_NUM_BINS = 256
_EMB_DIM = 128
_BATCH = 4096

_TOTAL = _BATCH * _STATE_DIM
_NC = 2
_NS = 16
_NW = _NC * _NS
_PER_W = _TOTAL // _NW
_CHUNK = 128
_NCHUNK = _PER_W // _CHUNK
_NBUF = 4
_K = 2
_LANES = 16
_VPC = _CHUNK // _LANES
_GRP = _STATE_DIM // _LANES


def _body(state_hbm, emb_hbm, smin_hbm, smax_hbm, out_hbm,
          state_v, idx_v, min_v, den_v, rows_v, *sems):
  gsem = sems[:_NBUF]
  ssem = sems[_NBUF:]
  wid = lax.axis_index("c") * _NS + lax.axis_index("s")
  base = wid * _PER_W

  pltpu.sync_copy(state_hbm.at[pl.ds(base, _PER_W)], state_v)
  pltpu.sync_copy(smin_hbm, min_v)
  pltpu.sync_copy(smax_hbm, den_v)
  for j in range(_GRP):
    sl = pl.ds(j * _LANES, _LANES)
    den_v[sl] = den_v[sl] - min_v[sl] + 1e-08

  def compute_row(r):
    for v in range(_VPC):
      j = v % _GRP
      sl = pl.ds(j * _LANES, _LANES)
      s = state_v[pl.ds(r * _CHUNK + v * _LANES, _LANES)]
      norm = (s - min_v[sl]) / den_v[sl]
      norm = jnp.clip(norm, 0.0, 1.0)
      bins = (norm * float(_NUM_BINS - 1)).astype(jnp.int32)
      dimoff = (lax.iota(jnp.int32, _LANES) + (j * _LANES)) * _NUM_BINS
      idx_v[r, pl.ds(v * _LANES, _LANES)] = bins + dimoff

  def g_copy(c, b):
    return pltpu.make_async_copy(emb_hbm.at[idx_v.at[c]], rows_v.at[b], gsem[b])

  def s_copy(c, b):
    return pltpu.make_async_copy(
        rows_v.at[b], out_hbm.at[pl.ds(base + c * _CHUNK, _CHUNK)], ssem[b])

  for c in range(_K):
    compute_row(c)
    g_copy(c, c % _NBUF).start()

  for c in range(_NBUF):
    compute_row(c + _K)
    if c >= _K:
      s_copy(c - _K, (c - _K) % _NBUF).wait()
    g_copy(c + _K, (c + _K) % _NBUF).start()
    g_copy(c, c % _NBUF).wait()
    s_copy(c, c % _NBUF).start()

  def outer(m, carry):
    o = m * _NBUF
    for b in range(_NBUF):
      c = o + b
      compute_row(c + _K)
      s_copy(c - _K, (b + _K) % _NBUF).wait()
      g_copy(c + _K, (b + _K) % _NBUF).start()
      g_copy(c, b).wait()
      s_copy(c, b).start()
    return carry

  lax.fori_loop(1, _NCHUNK // _NBUF - 1, outer, 0)

  o = _NCHUNK - _NBUF
  for b in range(_NBUF - _K):
    c = o + b
    compute_row(c + _K)
    s_copy(c - _K, (b + _K) % _NBUF).wait()
    g_copy(c + _K, (b + _K) % _NBUF).start()
    g_copy(c, b).wait()
    s_copy(c, b).start()
  for b in range(_NBUF - _K, _NBUF):
    c = o + b
    g_copy(c, b).wait()
    s_copy(c, b).start()
  for b in range(_NBUF):
    s_copy(o + b, b).wait()


_encode = pl.kernel(
    _body,
    out_type=jax.ShapeDtypeStruct((_TOTAL, _EMB_DIM), jnp.float32),
    mesh=plsc.VectorSubcoreMesh(
        core_axis_name="c", subcore_axis_name="s",
        num_cores=_NC, num_subcores=_NS),
    scratch_types=[
        pltpu.VMEM((_PER_W,), jnp.float32),
        pltpu.VMEM((_NCHUNK, _CHUNK), jnp.int32),
        pltpu.VMEM((_STATE_DIM,), jnp.float32),
        pltpu.VMEM((_STATE_DIM,), jnp.float32),
        pltpu.VMEM((_NBUF, _CHUNK, _EMB_DIM), jnp.float32),
    ] + [pltpu.SemaphoreType.DMA] * (2 * _NBUF),
)


@jax.jit
def kernel(state, embedding, state_min, state_max):
  out = _encode(state.reshape(_TOTAL), embedding, state_min, state_max)
  return out.reshape(_BATCH, _STATE_DIM, _EMB_DIM)
